# Initial kernel scaffold; baseline (speedup 1.0000x reference)
import jax
import jax.numpy as jnp
from jax.experimental import pallas as pl


def kernel(pc_idx, addr_idx, pc_table, addr_table, Wp1, bp1, Wp2, bp2, Wa1, ba1, Wa2, ba2):
    raise NotImplementedError("write your pallas kernel here")


# speedup vs baseline: 2.2144x; 2.2144x over previous
import functools

import jax
import jax.numpy as jnp
from jax import lax
from jax.experimental import pallas as pl
from jax.experimental.pallas import tpu as pltpu
from jax.experimental.pallas import tpu_sc as plsc

B = 16384
NW = 32
NB = B // NW
IPT = 4 * NB
L = 16


def _table_body(addr_t, pc_t, Wa1, ba1, Wa2, ba2, Wp1, bp1, Wp2, bp2, out_ref):
    a1 = jnp.maximum(
        jnp.dot(addr_t[...], Wa1[...], preferred_element_type=jnp.float32)
        + ba1[...], 0.0)
    a2 = jnp.maximum(
        jnp.dot(a1, Wa2[...], preferred_element_type=jnp.float32)
        + ba2[...], 0.0)
    p1 = jnp.maximum(
        jnp.dot(pc_t[...], Wp1[...], preferred_element_type=jnp.float32)
        + bp1[...], 0.0)
    p2 = jnp.maximum(
        jnp.dot(p1, Wp2[...], preferred_element_type=jnp.float32)
        + bp2[...], 0.0)
    out_ref[0:256, :] = a2
    out_ref[256:512, :] = p2


_table_call = pl.pallas_call(
    _table_body,
    out_shape=jax.ShapeDtypeStruct((512, 2), jnp.float32),
)


@functools.partial(
    pl.kernel,
    out_type=jax.ShapeDtypeStruct((8 * B * 2,), jnp.float32),
    mesh=plsc.VectorSubcoreMesh(core_axis_name="c", subcore_axis_name="s"),
    compiler_params=pltpu.CompilerParams(needs_layout_passes=False),
    scratch_types=[
        pltpu.VMEM((IPT,), jnp.int32),
        pltpu.VMEM((IPT,), jnp.int32),
        pltpu.VMEM((1024,), jnp.float32),
        pltpu.VMEM((8 * NB * 2,), jnp.float32),
    ],
)
def _sc_gather(addr_hbm, pc_hbm, tab_hbm, out_hbm, aidx_v, pidx_v, tab_v,
               stage_v):
    wid = lax.axis_index("s") * 2 + lax.axis_index("c")
    base = wid * IPT
    pltpu.sync_copy(addr_hbm.at[pl.ds(base, IPT)], aidx_v)
    pltpu.sync_copy(pc_hbm.at[pl.ds(base, IPT)], pidx_v)
    pltpu.sync_copy(tab_hbm, tab_v)

    iota = lax.broadcasted_iota(jnp.int32, (L,), 0)

    def make_body(idx_ref, tab_off, seg_off):
        def body(t, carry):
            j = t * L
            v_src = idx_ref[pl.ds(j, L)] * 2 + tab_off
            vj = j + iota
            v_i = lax.bitwise_and(vj, 3)
            v_b = lax.shift_right_logical(vj, 2)
            v_pos = (seg_off + v_i * NB + v_b) * 2
            c0 = plsc.load_gather(tab_v, [v_src])
            c1 = plsc.load_gather(tab_v, [v_src + 1])
            plsc.store_scatter(stage_v, [v_pos], c0)
            plsc.store_scatter(stage_v, [v_pos + 1], c1)
            return carry
        return body

    lax.fori_loop(0, IPT // L, make_body(aidx_v, 0, 0), 0)
    lax.fori_loop(0, IPT // L, make_body(pidx_v, 512, 4 * NB), 0)

    b0 = wid * NB
    for i in range(8):
        pltpu.sync_copy(stage_v.at[pl.ds(i * NB * 2, NB * 2)],
                        out_hbm.at[pl.ds((i * B + b0) * 2, NB * 2)])


def kernel(pc_idx, addr_idx, pc_table, addr_table,
           Wp1, bp1, Wp2, bp2, Wa1, ba1, Wa2, ba2):
    tab = _table_call(addr_table, pc_table,
                      Wa1, ba1.reshape(1, 8), Wa2, ba2.reshape(1, 2),
                      Wp1, bp1.reshape(1, 8), Wp2, bp2.reshape(1, 2))
    addr_flat = addr_idx.reshape(-1).astype(jnp.int32)
    pc_flat = pc_idx.reshape(-1).astype(jnp.int32)
    out = _sc_gather(addr_flat, pc_flat, tab.reshape(-1))
    return out.reshape(8 * B, 2)
